# baseline (speedup 1.0000x reference)
import jax
import jax.numpy as jnp
from jax import lax
from jax.experimental import pallas as pl
from jax.experimental.pallas import tpu as pltpu
from jax.experimental.pallas import tpu_sc as plsc

PCODE_DIM = 2380
FEATURE_DIM = 64
B, H, W = 1024, 19, 19
N = B * H * W
CHUNK = 512
NUM_CHUNKS = N // CHUNK

_info = plsc.get_sparse_core_info()
NC, NS, LANES = _info.num_cores, _info.num_subcores, _info.num_lanes
NW = NC * NS
FULL_TRIPS = NUM_CHUNKS // NW
REM = NUM_CHUNKS % NW


def _sc_body(table_hbm, pc0_hbm, pc1_hbm, bd0_hbm, bd1_hbm, out_hbm,
             pc0_v, pc1_v, bd0_v, bd1_v, idx0_v, idx1_v,
             rows0_v, rows1_v, sem0, sem1):
    wid = lax.axis_index("s") * NC + lax.axis_index("c")
    ntrips = FULL_TRIPS + jnp.where(wid < REM, 1, 0)

    def chunk_step(j, carry):
        chunk = wid + j * NW
        base = chunk * CHUNK

        pltpu.sync_copy(pc0_hbm.at[pl.ds(base, CHUNK)], pc0_v)
        pltpu.sync_copy(pc1_hbm.at[pl.ds(base, CHUNK)], pc1_v)
        pltpu.sync_copy(bd0_hbm.at[pl.ds(base, CHUNK)], bd0_v)
        pltpu.sync_copy(bd1_hbm.at[pl.ds(base, CHUNK)], bd1_v)

        def mask_step(i, c):
            s = pl.ds(i * LANES, LANES)
            occ0 = bd0_v[s] > 0.0
            occ1 = bd1_v[s] > 0.0
            idx0_v[s] = jnp.where(occ0, PCODE_DIM, pc0_v[s])
            idx1_v[s] = jnp.where(occ1, PCODE_DIM, pc1_v[s]) + (PCODE_DIM + 1)
            return c

        lax.fori_loop(0, CHUNK // LANES, mask_step, 0)

        cp0 = pltpu.async_copy(table_hbm.at[idx0_v], rows0_v, sem0)
        cp1 = pltpu.async_copy(table_hbm.at[idx1_v], rows1_v, sem1)
        cp0.wait()
        cp1.wait()

        def add_step(r, c):
            for k in range(FEATURE_DIM // LANES):
                s = pl.ds(k * LANES, LANES)
                plsc.addupdate(rows0_v.at[r, s], rows1_v[r, s])
            return c

        lax.fori_loop(0, CHUNK, add_step, 0)

        pltpu.sync_copy(rows0_v, out_hbm.at[chunk])
        return carry

    lax.fori_loop(0, ntrips, chunk_step, 0)


@jax.jit
def _sc_gather_sum(table, pc0, pc1, bd0, bd1):
    mesh = plsc.VectorSubcoreMesh(core_axis_name="c", subcore_axis_name="s")
    return pl.kernel(
        _sc_body,
        out_type=jax.ShapeDtypeStruct((NUM_CHUNKS, CHUNK, FEATURE_DIM),
                                      jnp.float32),
        mesh=mesh,
        compiler_params=pltpu.CompilerParams(use_tc_tiling_on_sc=False),
        scratch_types=[
            pltpu.VMEM((CHUNK,), jnp.int32),
            pltpu.VMEM((CHUNK,), jnp.int32),
            pltpu.VMEM((CHUNK,), jnp.float32),
            pltpu.VMEM((CHUNK,), jnp.float32),
            pltpu.VMEM((CHUNK,), jnp.int32),
            pltpu.VMEM((CHUNK,), jnp.int32),
            pltpu.VMEM((CHUNK, FEATURE_DIM), jnp.float32),
            pltpu.VMEM((CHUNK, FEATURE_DIM), jnp.float32),
            pltpu.SemaphoreType.DMA,
            pltpu.SemaphoreType.DMA,
        ],
    )(table, pc0, pc1, bd0, bd1)


def kernel(sparse_feature_input, board_input, sparse_feature_dim, pcode_table):
    del sparse_feature_dim
    pc0 = sparse_feature_input[:, 10].reshape(N)
    pc1 = sparse_feature_input[:, 11].reshape(N)
    bd0 = board_input[:, 0].reshape(N)
    bd1 = board_input[:, 1].reshape(N)
    flat = _sc_gather_sum(pcode_table, pc0, pc1, bd0, bd1)
    feat = flat.reshape(B, H, W, FEATURE_DIM)
    return jnp.transpose(feat, (0, 3, 1, 2))
